# baseline (speedup 1.0000x reference)
import functools

import jax
import jax.numpy as jnp
from jax import lax
from jax.experimental import pallas as pl
from jax.experimental.pallas import tpu as pltpu
from jax.experimental.pallas import tpu_sc as plsc

VOCAB = 100000
DIM = 64
BATCH = 1024
CTX = 200

NC = 2
NS = 16
NW = NC * NS
L = 16

BPW = BATCH // NW
CHUNK_A = 104
CHUNK_B = CTX - CHUNK_A
DV = DIM // L


def _sc_pool(table, idx_flat):
  @functools.partial(
      pl.kernel,
      out_type=jax.ShapeDtypeStruct((BATCH, DIM), jnp.float32),
      mesh=plsc.VectorSubcoreMesh(core_axis_name="c", subcore_axis_name="s"),
      compiler_params=pltpu.CompilerParams(use_tc_tiling_on_sc=False),
      scratch_types=[
          pltpu.VMEM((BPW * CTX,), jnp.int32),
          pltpu.VMEM((CTX, DIM), jnp.float32),
          pltpu.VMEM((BPW, DIM), jnp.float32),
          pltpu.SemaphoreType.DMA,
      ],
  )
  def sc_kernel(table_hbm, idx_hbm, out_hbm, idx_v, rows_v, pooled_v, sem):
    wid = lax.axis_index("s") * NC + lax.axis_index("c")
    base = wid * BPW
    pltpu.sync_copy(idx_hbm.at[pl.ds(base * CTX, BPW * CTX)], idx_v)

    def body_b(b, carry):
      off = b * CTX
      cp1 = pltpu.async_copy(
          table_hbm.at[idx_v.at[pl.ds(off, CHUNK_A)]],
          rows_v.at[pl.ds(0, CHUNK_A)], sem)
      cp2 = pltpu.async_copy(
          table_hbm.at[idx_v.at[pl.ds(off + CHUNK_A, CHUNK_B)]],
          rows_v.at[pl.ds(CHUNK_A, CHUNK_B)], sem)
      cp1.wait()
      cp2.wait()

      def body_j(j, accs):
        return tuple(accs[k] + rows_v[j, pl.ds(k * L, L)] for k in range(DV))

      accs = lax.fori_loop(
          0, CTX, body_j,
          tuple(jnp.zeros((L,), jnp.float32) for _ in range(DV)))
      for k in range(DV):
        pooled_v[b, pl.ds(k * L, L)] = accs[k] * (1.0 / CTX)
      return carry

    lax.fori_loop(0, BPW, body_b, 0)
    pltpu.sync_copy(pooled_v, out_hbm.at[pl.ds(base, BPW)])

  return sc_kernel(table, idx_flat)


V_TILE = 4096
V_GRID = (VOCAB + V_TILE - 1) // V_TILE


def _mm_body(x_ref, w_ref, b_ref, o_ref):
  o_ref[...] = lax.dot_general(
      x_ref[...], w_ref[...],
      dimension_numbers=(((1,), (1,)), ((), ())),
      preferred_element_type=jnp.float32) + b_ref[...]


def _tc_project(pooled, W, b2d):
  return pl.pallas_call(
      _mm_body,
      grid=(V_GRID,),
      in_specs=[
          pl.BlockSpec((BATCH, DIM), lambda i: (0, 0)),
          pl.BlockSpec((V_TILE, DIM), lambda i: (i, 0)),
          pl.BlockSpec((1, V_TILE), lambda i: (0, i)),
      ],
      out_specs=pl.BlockSpec((BATCH, V_TILE), lambda i: (0, i)),
      out_shape=jax.ShapeDtypeStruct((BATCH, VOCAB), jnp.float32),
  )(pooled, W, b2d)


def kernel(emb_table, W, b, inputs):
  pooled = _sc_pool(emb_table, inputs.reshape(-1))
  return _tc_project(pooled, W, b.reshape(1, VOCAB))


# speedup vs baseline: 2.1847x; 2.1847x over previous
import functools

import jax
import jax.numpy as jnp
from jax import lax
from jax.experimental import pallas as pl
from jax.experimental.pallas import tpu as pltpu
from jax.experimental.pallas import tpu_sc as plsc

VOCAB = 100000
DIM = 64
BATCH = 1024
CTX = 200

NC = 2
NS = 16
NW = NC * NS
L = 16

BPW = BATCH // NW
CHUNK_A = 104
CHUNK_B = CTX - CHUNK_A
DV = DIM // L


def _sc_pool(table, idx_flat):
  @functools.partial(
      pl.kernel,
      out_type=jax.ShapeDtypeStruct((BATCH, DIM), jnp.float32),
      mesh=plsc.VectorSubcoreMesh(core_axis_name="c", subcore_axis_name="s"),
      compiler_params=pltpu.CompilerParams(use_tc_tiling_on_sc=False),
      scratch_types=[
          pltpu.VMEM((BPW * CTX,), jnp.int32),
          pltpu.VMEM((CTX, DIM), jnp.float32),
          pltpu.VMEM((BPW, DIM), jnp.float32),
          pltpu.SemaphoreType.DMA,
      ],
  )
  def sc_kernel(table_hbm, idx_hbm, out_hbm, idx_v, rows_v, pooled_v, sem):
    wid = lax.axis_index("s") * NC + lax.axis_index("c")
    base = wid * BPW
    pltpu.sync_copy(idx_hbm.at[pl.ds(base * CTX, BPW * CTX)], idx_v)

    def body_b(b, carry):
      off = b * CTX
      cp1 = pltpu.async_copy(
          table_hbm.at[idx_v.at[pl.ds(off, CHUNK_A)]],
          rows_v.at[pl.ds(0, CHUNK_A)], sem)
      cp2 = pltpu.async_copy(
          table_hbm.at[idx_v.at[pl.ds(off + CHUNK_A, CHUNK_B)]],
          rows_v.at[pl.ds(CHUNK_A, CHUNK_B)], sem)
      cp1.wait()
      cp2.wait()

      def body_j(j, accs):
        return tuple(accs[k] + rows_v[j, pl.ds(k * L, L)] for k in range(DV))

      accs = lax.fori_loop(
          0, CTX, body_j,
          tuple(jnp.zeros((L,), jnp.float32) for _ in range(DV)))
      for k in range(DV):
        pooled_v[b, pl.ds(k * L, L)] = accs[k] * (1.0 / CTX)
      return carry

    lax.fori_loop(0, BPW, body_b, 0)
    pltpu.sync_copy(pooled_v, out_hbm.at[pl.ds(base, BPW)])

  return sc_kernel(table, idx_flat)


V_TILE = 4096
V_GRID = (VOCAB + V_TILE - 1) // V_TILE


def _mm_body(wt_ref, x_ref, b_ref, o_ref):
  o_ref[...] = lax.dot_general(
      wt_ref[...], x_ref[...],
      dimension_numbers=(((0,), (1,)), ((), ())),
      preferred_element_type=jnp.float32) + b_ref[...]


def _tc_project(pooled, Wt, b2d):
  return pl.pallas_call(
      _mm_body,
      grid=(V_GRID,),
      in_specs=[
          pl.BlockSpec((DIM, V_TILE), lambda i: (0, i)),
          pl.BlockSpec((BATCH, DIM), lambda i: (0, 0)),
          pl.BlockSpec((V_TILE, 1), lambda i: (i, 0)),
      ],
      out_specs=pl.BlockSpec((V_TILE, BATCH), lambda i: (i, 0)),
      out_shape=jax.ShapeDtypeStruct((VOCAB, BATCH), jnp.float32),
  )(Wt, pooled, b2d)


def kernel(emb_table, W, b, inputs):
  pooled = _sc_pool(emb_table, inputs.reshape(-1))
  out_t = _tc_project(pooled, W.T, b.reshape(VOCAB, 1))
  return out_t.T


# speedup vs baseline: 2.3326x; 1.0677x over previous
import functools

import jax
import jax.numpy as jnp
from jax import lax
from jax.experimental import pallas as pl
from jax.experimental.pallas import tpu as pltpu
from jax.experimental.pallas import tpu_sc as plsc

VOCAB = 100000
DIM = 64
BATCH = 1024
CTX = 200

NC = 2
NS = 16
NW = NC * NS
L = 16

BPW = BATCH // NW
CHUNK_A = 104
CHUNK_B = CTX - CHUNK_A
DV = DIM // L


def _sc_pool(table, idx_flat):
  @functools.partial(
      pl.kernel,
      out_type=jax.ShapeDtypeStruct((BATCH, DIM), jnp.float32),
      mesh=plsc.VectorSubcoreMesh(core_axis_name="c", subcore_axis_name="s"),
      compiler_params=pltpu.CompilerParams(use_tc_tiling_on_sc=False),
      scratch_types=[
          pltpu.VMEM((BPW * CTX,), jnp.int32),
          pltpu.VMEM((2, CTX, DIM), jnp.float32),
          pltpu.VMEM((BPW, DIM), jnp.float32),
          pltpu.SemaphoreType.DMA,
          pltpu.SemaphoreType.DMA,
      ],
  )
  def sc_kernel(table_hbm, idx_hbm, out_hbm, idx_v, rows_v, pooled_v,
                sem0, sem1):
    wid = lax.axis_index("s") * NC + lax.axis_index("c")
    base = wid * BPW
    sems = (sem0, sem1)
    pltpu.sync_copy(idx_hbm.at[pl.ds(base * CTX, BPW * CTX)], idx_v)

    def issue(b, buf):
      off = b * CTX
      pltpu.async_copy(
          table_hbm.at[idx_v.at[pl.ds(off, CHUNK_A)]],
          rows_v.at[buf, pl.ds(0, CHUNK_A)], sems[buf])
      pltpu.async_copy(
          table_hbm.at[idx_v.at[pl.ds(off + CHUNK_A, CHUNK_B)]],
          rows_v.at[buf, pl.ds(CHUNK_A, CHUNK_B)], sems[buf])

    def drain(b, buf):
      off = b * CTX
      pltpu.make_async_copy(
          table_hbm.at[idx_v.at[pl.ds(off, CHUNK_A)]],
          rows_v.at[buf, pl.ds(0, CHUNK_A)], sems[buf]).wait()
      pltpu.make_async_copy(
          table_hbm.at[idx_v.at[pl.ds(off + CHUNK_A, CHUNK_B)]],
          rows_v.at[buf, pl.ds(CHUNK_A, CHUNK_B)], sems[buf]).wait()

    def reduce_into(b, buf):
      def body_j(j, accs):
        return tuple(
            accs[k] + rows_v[buf, j, pl.ds(k * L, L)] for k in range(DV))

      accs = lax.fori_loop(
          0, CTX, body_j,
          tuple(jnp.zeros((L,), jnp.float32) for _ in range(DV)),
          unroll=4)
      for k in range(DV):
        pooled_v[b, pl.ds(k * L, L)] = accs[k] * (1.0 / CTX)

    issue(0, 0)

    def body_pair(i, carry):
      b0 = 2 * i
      drain(b0, 0)
      issue(b0 + 1, 1)
      reduce_into(b0, 0)
      drain(b0 + 1, 1)

      @pl.when(b0 + 2 < BPW)
      def _():
        issue(b0 + 2, 0)

      reduce_into(b0 + 1, 1)
      return carry

    lax.fori_loop(0, BPW // 2, body_pair, 0)
    pltpu.sync_copy(pooled_v, out_hbm.at[pl.ds(base, BPW)])

  return sc_kernel(table, idx_flat)


V_TILE = 4096
V_GRID = (VOCAB + V_TILE - 1) // V_TILE


def _mm_body(wt_ref, x_ref, b_ref, o_ref):
  o_ref[...] = lax.dot_general(
      wt_ref[...], x_ref[...],
      dimension_numbers=(((0,), (1,)), ((), ())),
      preferred_element_type=jnp.float32) + b_ref[...]


def _tc_project(pooled, Wt, b2d):
  return pl.pallas_call(
      _mm_body,
      grid=(V_GRID,),
      in_specs=[
          pl.BlockSpec((DIM, V_TILE), lambda i: (0, i)),
          pl.BlockSpec((BATCH, DIM), lambda i: (0, 0)),
          pl.BlockSpec((V_TILE, 1), lambda i: (i, 0)),
      ],
      out_specs=pl.BlockSpec((V_TILE, BATCH), lambda i: (i, 0)),
      out_shape=jax.ShapeDtypeStruct((VOCAB, BATCH), jnp.float32),
  )(Wt, pooled, b2d)


def kernel(emb_table, W, b, inputs):
  pooled = _sc_pool(emb_table, inputs.reshape(-1))
  out_t = _tc_project(pooled, W.T, b.reshape(VOCAB, 1))
  return out_t.T
